# baseline (speedup 1.0000x reference)
import functools

import jax
import jax.numpy as jnp
from jax import lax
from jax.experimental import pallas as pl
from jax.experimental.pallas import tpu as pltpu
from jax.experimental.pallas import tpu_sc as plsc

_B = 4096
_D = 128
_NSQ = 64
_NTOK = _NSQ + 7
_STRIDE = 648
_MAT_OFF = 448
_CTX_OFF = 512
_NTOT = _NSQ * _STRIDE
_EPS = 1e-5

_NW = 32
_PTOK = 72
_NCH = 4
_CBRD = _B // _NCH
_CBW = _CBRD // _NW
_NB = 4
_NSUP = _CBW // (2 * _NB)
_RB = 64
_RSTEPS = _CBRD // _RB


def _table_body(pt_ref, cw_ref, cb_ref, sq_ref, aw_ref, ab_ref, ps_ref,
                iso_ref, db_ref, tt_ref, ct_ref, et_ref, pht_ref, ckt_ref,
                mt_ref, mb_ref, mw_ref, mvb_ref, g_ref, bt_ref, out_ref):
    g = g_ref[...]
    bb = bt_ref[...]

    def ln(x):
        m = jnp.mean(x, axis=-1, keepdims=True)
        xc = x - m
        v = jnp.mean(xc * xc, axis=-1, keepdims=True)
        return xc * lax.rsqrt(v + _EPS) * g + bb

    pt = pt_ref[...]
    cb = cb_ref[...]
    p0 = jnp.dot(pt, cw_ref[0], preferred_element_type=jnp.float32) + cb[0:1]
    p1 = jnp.dot(pt, cw_ref[1], preferred_element_type=jnp.float32) + cb[1:2]
    proj14 = jnp.concatenate([p0[:, None, :], p1[:, None, :]], axis=1)
    proj14 = proj14.reshape(14, 1, _D)

    aw = aw_ref[...]
    ab = ab_ref[...]
    ps = ps_ref[...]
    iso = iso_ref[...]
    db = db_ref[...]
    f = lax.broadcasted_iota(jnp.int32, (32, 1), 0)
    bit = lambda k: ((f >> k) & 1).astype(jnp.float32)
    base0 = aw[0:1] + ab[0:1] + ps[0:1] + iso[0:1] + db[0:1]
    flag32 = (base0
              + bit(4) * (aw[1:2] - aw[0:1])
              + bit(3) * (ab[1:2] - ab[0:1])
              + bit(2) * (ps[1:2] - ps[0:1])
              + bit(1) * (iso[1:2] - iso[0:1])
              + bit(0) * (db[1:2] - db[0:1]))

    fused = (proj14 + flag32.reshape(1, 32, _D)).reshape(448, _D)
    out_ref[0:448] = ln(fused + sq_ref[0])

    m = mb_ref[0]
    mat = lax.dot_general(m, mw_ref[...], (((0,), (0,)), ((), ())),
                          preferred_element_type=jnp.float32)
    out_ref[_MAT_OFF:_CTX_OFF] = ln(mat + mvb_ref[...])

    ctx = jnp.concatenate([tt_ref[...], ct_ref[...], et_ref[...],
                           pht_ref[...], ckt_ref[...], mt_ref[...]], axis=0)
    out_ref[_CTX_OFF:_CTX_OFF + 132] = ln(ctx)
    out_ref[644:_STRIDE] = jnp.zeros((4, _D), jnp.float32)


def _gather_body(table_hbm, idx_hbm, out_hbm, idx_v, bufs, gs0, gs1, ss0, ss1):
    wid = lax.axis_index("s") * 2 + lax.axis_index("c")
    base = wid * _CBW
    pltpu.sync_copy(idx_hbm.at[pl.ds(base * _PTOK, _CBW * _PTOK)], idx_v)
    gsem = (gs0, gs1)
    ssem = (ss0, ss1)

    def issue_gathers(g, p):
        for b in range(_NB):
            j = g * _NB + b
            pltpu.async_copy(
                table_hbm.at[idx_v.at[pl.ds(j * _PTOK, _PTOK)]],
                bufs.at[p, pl.ds(b * _PTOK, _PTOK)], gsem[p])

    def drain_gathers(p):
        for b in range(_NB):
            pltpu.make_async_copy(out_hbm.at[pl.ds(0, _PTOK)],
                                  bufs.at[p, pl.ds(b * _PTOK, _PTOK)],
                                  gsem[p]).wait()

    def issue_store(g, p):
        pltpu.async_copy(
            bufs.at[p],
            out_hbm.at[pl.ds((base + g * _NB) * _PTOK, _NB * _PTOK)], ssem[p])

    def drain_store(p):
        pltpu.make_async_copy(bufs.at[p],
                              out_hbm.at[pl.ds(0, _NB * _PTOK)],
                              ssem[p]).wait()

    issue_gathers(0, 0)

    def super_body(si, carry):
        g0 = si * 2

        @pl.when(si > 0)
        def _():
            drain_store(1)
        issue_gathers(g0 + 1, 1)
        drain_gathers(0)
        issue_store(g0, 0)
        drain_gathers(1)
        issue_store(g0 + 1, 1)

        @pl.when(si < _NSUP - 1)
        def _():
            drain_store(0)
            issue_gathers(g0 + 2, 0)
        return carry

    lax.fori_loop(0, _NSUP, super_body, 0)
    drain_store(0)
    drain_store(1)


def _sc_gather_chunk(table, idx_chunk):
    f = functools.partial(
        pl.kernel,
        mesh=plsc.VectorSubcoreMesh(core_axis_name="c", subcore_axis_name="s"),
        out_type=jax.ShapeDtypeStruct((_CBRD * _PTOK, _D), jnp.float32),
        scratch_types=[
            pltpu.VMEM((_CBW * _PTOK,), jnp.int32),
            pltpu.VMEM((2, _NB * _PTOK, _D), jnp.float32),
            pltpu.SemaphoreType.DMA,
            pltpu.SemaphoreType.DMA,
            pltpu.SemaphoreType.DMA,
            pltpu.SemaphoreType.DMA,
        ],
    )(_gather_body)
    return f(table, idx_chunk)


def _repack_first_body(src_ref, out_ref):
    out_ref[...] = src_ref[...].reshape(_RB, _PTOK, _D)[:, :_NTOK, :]


def _repack_next_body(src_ref, prev_ref, out_ref):
    out_ref[...] = src_ref[...].reshape(_RB, _PTOK, _D)[:, :_NTOK, :]


def _repack(chunk_flat, prev_out, k):
    src_spec = pl.BlockSpec((_RB * _PTOK, _D), lambda i: (i, 0))
    out_spec = pl.BlockSpec((_RB, _NTOK, _D),
                            lambda i, k=k: (k * _RSTEPS + i, 0, 0))
    out_shape = jax.ShapeDtypeStruct((_B, _NTOK, _D), jnp.float32)
    if prev_out is None:
        return pl.pallas_call(
            _repack_first_body,
            grid=(_RSTEPS,),
            in_specs=[src_spec],
            out_specs=out_spec,
            out_shape=out_shape,
        )(chunk_flat)
    return pl.pallas_call(
        _repack_next_body,
        grid=(_RSTEPS,),
        in_specs=[src_spec, pl.BlockSpec((1, _NTOK, _D), lambda i: (0, 0, 0))],
        out_specs=out_spec,
        out_shape=out_shape,
        input_output_aliases={1: 0},
    )(chunk_flat, prev_out)


@jax.jit
def kernel(piece_ids, color_ids, white_attacks, black_attacks, is_passed,
           is_isolated, is_doubled, material_balance, game_phase, is_check,
           mobility, turn, castling, ep, piece_table, color_W, color_b,
           square_table, turn_table, castling_table, ep_table, attack_w_table,
           attack_b_table, passed_table, isolated_table, doubled_table,
           material_W, material_b, phase_table, check_table, mobility_table,
           ln_gamma, ln_beta):
    full = lambda a: pl.BlockSpec(a.shape, lambda i: (0,) * a.ndim)

    sq_r = square_table.reshape(_NSQ, 1, _D)
    mb_r = material_balance.reshape(_NSQ, 1, 64)
    mw_r = material_W.reshape(1, _D)
    mvb_r = material_b.reshape(1, _D)
    g_r = ln_gamma.reshape(1, _D)
    b_r = ln_beta.reshape(1, _D)

    table = pl.pallas_call(
        _table_body,
        grid=(_NSQ,),
        in_specs=[
            full(piece_table), full(color_W), full(color_b),
            pl.BlockSpec((1, 1, _D), lambda i: (i, 0, 0)),
            full(attack_w_table), full(attack_b_table), full(passed_table),
            full(isolated_table), full(doubled_table),
            full(turn_table), full(castling_table), full(ep_table),
            full(phase_table), full(check_table), full(mobility_table),
            pl.BlockSpec((1, 1, 64), lambda i: (i, 0, 0)),
            full(mw_r), full(mvb_r), full(g_r), full(b_r),
        ],
        out_specs=pl.BlockSpec((_STRIDE, _D), lambda i: (i, 0)),
        out_shape=jax.ShapeDtypeStruct((_NTOT, _D), jnp.float32),
    )(piece_table, color_W, color_b, sq_r, attack_w_table, attack_b_table,
      passed_table, isolated_table, doubled_table, turn_table, castling_table,
      ep_table, phase_table, check_table, mobility_table, mb_r, mw_r, mvb_r,
      g_r, b_r)

    cidx = (piece_ids * 64 + color_ids * 32 + white_attacks * 16
            + black_attacks * 8 + is_passed * 4 + is_isolated * 2 + is_doubled)
    t_iota = jnp.arange(_NSQ, dtype=jnp.int32)[None, :]
    idx_sq = t_iota * _STRIDE + cidx
    b_iota = jnp.arange(_B, dtype=jnp.int32)
    blk_base = (b_iota // 64) * _STRIDE
    cbase = blk_base + _CTX_OFF
    idx_ctx = jnp.stack([
        cbase + turn,
        cbase + 2 + castling,
        cbase + 18 + ep,
        blk_base + _MAT_OFF + (b_iota % 64),
        cbase + 27 + game_phase,
        cbase + 30 + is_check,
        cbase + 32 + jnp.clip(mobility, 0, 99),
    ], axis=1)
    pad = jnp.zeros((_B, _PTOK - _NTOK), jnp.int32)
    idx_all = jnp.concatenate([idx_sq, idx_ctx, pad], axis=1).reshape(-1)
    idx_all = idx_all.astype(jnp.int32)

    chunks = [
        _sc_gather_chunk(
            table,
            lax.slice(idx_all, (k * _CBRD * _PTOK,), ((k + 1) * _CBRD * _PTOK,)))
        for k in range(_NCH)
    ]
    out = None
    for k in range(_NCH):
        out = _repack(chunks[k], out, k)
    return out


# speedup vs baseline: 2.3449x; 2.3449x over previous
import functools

import jax
import jax.numpy as jnp
from jax import lax
from jax.experimental import pallas as pl
from jax.experimental.pallas import tpu as pltpu
from jax.experimental.pallas import tpu_sc as plsc

_B = 4096
_D = 128
_NSQ = 64
_NTOK = _NSQ + 7
_STRIDE = 648
_MAT_OFF = 448
_CTX_OFF = 512
_NTOT = _NSQ * _STRIDE
_EPS = 1e-5

_NW = 32
_BPW = _B // _NW
_PTOK = 72
_NB = 4
_NSUP = _BPW // (2 * _NB)


def _table_body(pt_ref, cw_ref, cb_ref, sq_ref, aw_ref, ab_ref, ps_ref,
                iso_ref, db_ref, tt_ref, ct_ref, et_ref, pht_ref, ckt_ref,
                mt_ref, mb_ref, mw_ref, mvb_ref, g_ref, bt_ref, out_ref,
                fused_scr, ctx_scr):
    g = g_ref[...]
    bb = bt_ref[...]

    def ln(x):
        m = jnp.mean(x, axis=-1, keepdims=True)
        xc = x - m
        v = jnp.mean(xc * xc, axis=-1, keepdims=True)
        return xc * lax.rsqrt(v + _EPS) * g + bb

    i = pl.program_id(0)

    @pl.when(i == 0)
    def _():
        pt = pt_ref[...]
        cb = cb_ref[...]
        p0 = (jnp.dot(pt, cw_ref[0], preferred_element_type=jnp.float32)
              + cb[0:1])
        p1 = (jnp.dot(pt, cw_ref[1], preferred_element_type=jnp.float32)
              + cb[1:2])
        proj14 = jnp.concatenate([p0[:, None, :], p1[:, None, :]], axis=1)
        proj14 = proj14.reshape(14, 1, _D)

        aw = aw_ref[...]
        ab = ab_ref[...]
        ps = ps_ref[...]
        iso = iso_ref[...]
        db = db_ref[...]
        f = lax.broadcasted_iota(jnp.int32, (32, 1), 0)
        bit = lambda k: ((f >> k) & 1).astype(jnp.float32)
        base0 = aw[0:1] + ab[0:1] + ps[0:1] + iso[0:1] + db[0:1]
        flag32 = (base0
                  + bit(4) * (aw[1:2] - aw[0:1])
                  + bit(3) * (ab[1:2] - ab[0:1])
                  + bit(2) * (ps[1:2] - ps[0:1])
                  + bit(1) * (iso[1:2] - iso[0:1])
                  + bit(0) * (db[1:2] - db[0:1]))

        fused_scr[...] = (proj14 + flag32.reshape(1, 32, _D)).reshape(448, _D)

        ctx = jnp.concatenate([tt_ref[...], ct_ref[...], et_ref[...],
                               pht_ref[...], ckt_ref[...], mt_ref[...]],
                              axis=0)
        ctx_scr[...] = ln(ctx)

    out_ref[0:448] = ln(fused_scr[...] + sq_ref[0])

    m = mb_ref[0]
    mat = lax.dot_general(m, mw_ref[...], (((0,), (0,)), ((), ())),
                          preferred_element_type=jnp.float32)
    out_ref[_MAT_OFF:_CTX_OFF] = ln(mat + mvb_ref[...])

    out_ref[_CTX_OFF:_CTX_OFF + 132] = ctx_scr[...]
    out_ref[644:_STRIDE] = jnp.zeros((4, _D), jnp.float32)


def _gather_body(table_hbm, idx_hbm, out_hbm, idx_v, bufs, gs0, gs1, ss0, ss1):
    wid = lax.axis_index("s") * 2 + lax.axis_index("c")
    base = wid * _BPW
    pltpu.sync_copy(idx_hbm.at[pl.ds(base * _PTOK, _BPW * _PTOK)], idx_v)
    gsem = (gs0, gs1)
    ssem = (ss0, ss1)

    def issue_gathers(g, p):
        for b in range(_NB):
            j = g * _NB + b
            pltpu.async_copy(
                table_hbm.at[idx_v.at[pl.ds(j * _PTOK, _NTOK)]],
                bufs.at[p, b], gsem[p])

    def drain_gathers(p):
        for b in range(_NB):
            pltpu.make_async_copy(out_hbm.at[base], bufs.at[p, b],
                                  gsem[p]).wait()

    def issue_store(g, p):
        pltpu.async_copy(bufs.at[p], out_hbm.at[pl.ds(base + g * _NB, _NB)],
                         ssem[p])

    def drain_store(p):
        pltpu.make_async_copy(bufs.at[p], out_hbm.at[pl.ds(base, _NB)],
                              ssem[p]).wait()

    issue_gathers(0, 0)

    def super_body(si, carry):
        g0 = si * 2

        @pl.when(si > 0)
        def _():
            drain_store(1)
        issue_gathers(g0 + 1, 1)
        drain_gathers(0)
        issue_store(g0, 0)
        drain_gathers(1)
        issue_store(g0 + 1, 1)

        @pl.when(si < _NSUP - 1)
        def _():
            drain_store(0)
            issue_gathers(g0 + 2, 0)
        return carry

    lax.fori_loop(0, _NSUP, super_body, 0)
    drain_store(0)
    drain_store(1)


def _sc_gather(table, idx_all):
    f = functools.partial(
        pl.kernel,
        mesh=plsc.VectorSubcoreMesh(core_axis_name="c", subcore_axis_name="s"),
        out_type=jax.ShapeDtypeStruct((_B, _NTOK, _D), jnp.float32),
        scratch_types=[
            pltpu.VMEM((_BPW * _PTOK,), jnp.int32),
            pltpu.VMEM((2, _NB, _NTOK, _D), jnp.float32),
            pltpu.SemaphoreType.DMA,
            pltpu.SemaphoreType.DMA,
            pltpu.SemaphoreType.DMA,
            pltpu.SemaphoreType.DMA,
        ],
    )(_gather_body)
    return f(table, idx_all)


@jax.jit
def kernel(piece_ids, color_ids, white_attacks, black_attacks, is_passed,
           is_isolated, is_doubled, material_balance, game_phase, is_check,
           mobility, turn, castling, ep, piece_table, color_W, color_b,
           square_table, turn_table, castling_table, ep_table, attack_w_table,
           attack_b_table, passed_table, isolated_table, doubled_table,
           material_W, material_b, phase_table, check_table, mobility_table,
           ln_gamma, ln_beta):
    full = lambda a: pl.BlockSpec(a.shape, lambda i: (0,) * a.ndim)

    sq_r = square_table.reshape(_NSQ, 1, _D)
    mb_r = material_balance.reshape(_NSQ, 1, 64)
    mw_r = material_W.reshape(1, _D)
    mvb_r = material_b.reshape(1, _D)
    g_r = ln_gamma.reshape(1, _D)
    b_r = ln_beta.reshape(1, _D)

    table = pl.pallas_call(
        _table_body,
        grid=(_NSQ,),
        in_specs=[
            full(piece_table), full(color_W), full(color_b),
            pl.BlockSpec((1, 1, _D), lambda i: (i, 0, 0)),
            full(attack_w_table), full(attack_b_table), full(passed_table),
            full(isolated_table), full(doubled_table),
            full(turn_table), full(castling_table), full(ep_table),
            full(phase_table), full(check_table), full(mobility_table),
            pl.BlockSpec((1, 1, 64), lambda i: (i, 0, 0)),
            full(mw_r), full(mvb_r), full(g_r), full(b_r),
        ],
        out_specs=pl.BlockSpec((_STRIDE, _D), lambda i: (i, 0)),
        out_shape=jax.ShapeDtypeStruct((_NTOT, _D), jnp.float32),
        scratch_shapes=[
            pltpu.VMEM((448, _D), jnp.float32),
            pltpu.VMEM((132, _D), jnp.float32),
        ],
    )(piece_table, color_W, color_b, sq_r, attack_w_table, attack_b_table,
      passed_table, isolated_table, doubled_table, turn_table, castling_table,
      ep_table, phase_table, check_table, mobility_table, mb_r, mw_r, mvb_r,
      g_r, b_r)

    cidx = (piece_ids * 64 + color_ids * 32 + white_attacks * 16
            + black_attacks * 8 + is_passed * 4 + is_isolated * 2 + is_doubled)
    t_iota = jnp.arange(_NSQ, dtype=jnp.int32)[None, :]
    idx_sq = t_iota * _STRIDE + cidx
    b_iota = jnp.arange(_B, dtype=jnp.int32)
    blk_base = (b_iota // 64) * _STRIDE
    cbase = blk_base + _CTX_OFF
    idx_ctx = jnp.stack([
        cbase + turn,
        cbase + 2 + castling,
        cbase + 18 + ep,
        blk_base + _MAT_OFF + (b_iota % 64),
        cbase + 27 + game_phase,
        cbase + 30 + is_check,
        cbase + 32 + jnp.clip(mobility, 0, 99),
    ], axis=1)
    pad = jnp.zeros((_B, _PTOK - _NTOK), jnp.int32)
    idx_all = jnp.concatenate([idx_sq, idx_ctx, pad], axis=1).reshape(-1)
    idx_all = idx_all.astype(jnp.int32)

    return _sc_gather(table, idx_all)


# speedup vs baseline: 2.5160x; 1.0730x over previous
import functools

import jax
import jax.numpy as jnp
from jax import lax
from jax.experimental import pallas as pl
from jax.experimental.pallas import tpu as pltpu
from jax.experimental.pallas import tpu_sc as plsc

_B = 4096
_D = 128
_NSQ = 64
_NTOK = _NSQ + 7
_STRIDE = 648
_MAT_OFF = 448
_CTX_OFF = 512
_NTOT = _NSQ * _STRIDE
_EPS = 1e-5

_SQS = 8
_TSTEPS = _NSQ // _SQS

_NW = 32
_BPW = _B // _NW
_PTOK = 72
_NB = 4
_NSUP = _BPW // (2 * _NB)


def _table_body(pt_ref, cw_ref, cb_ref, sq_ref, aw_ref, ab_ref, ps_ref,
                iso_ref, db_ref, tt_ref, ct_ref, et_ref, pht_ref, ckt_ref,
                mt_ref, mb_ref, mw_ref, mvb_ref, g_ref, bt_ref, out_ref,
                fused_scr, ctx_scr):
    g = g_ref[...]
    bb = bt_ref[...]

    def ln(x):
        m = jnp.mean(x, axis=-1, keepdims=True)
        xc = x - m
        v = jnp.mean(xc * xc, axis=-1, keepdims=True)
        return xc * lax.rsqrt(v + _EPS) * g + bb

    i = pl.program_id(0)

    @pl.when(i == 0)
    def _():
        pt = pt_ref[...]
        cb = cb_ref[...]
        p0 = (jnp.dot(pt, cw_ref[0], preferred_element_type=jnp.float32)
              + cb[0:1])
        p1 = (jnp.dot(pt, cw_ref[1], preferred_element_type=jnp.float32)
              + cb[1:2])
        proj14 = jnp.concatenate([p0[:, None, :], p1[:, None, :]], axis=1)
        proj14 = proj14.reshape(14, 1, _D)

        aw = aw_ref[...]
        ab = ab_ref[...]
        ps = ps_ref[...]
        iso = iso_ref[...]
        db = db_ref[...]
        f = lax.broadcasted_iota(jnp.int32, (32, 1), 0)
        bit = lambda k: ((f >> k) & 1).astype(jnp.float32)
        base0 = aw[0:1] + ab[0:1] + ps[0:1] + iso[0:1] + db[0:1]
        flag32 = (base0
                  + bit(4) * (aw[1:2] - aw[0:1])
                  + bit(3) * (ab[1:2] - ab[0:1])
                  + bit(2) * (ps[1:2] - ps[0:1])
                  + bit(1) * (iso[1:2] - iso[0:1])
                  + bit(0) * (db[1:2] - db[0:1]))

        fused_scr[...] = (proj14 + flag32.reshape(1, 32, _D)).reshape(448, _D)

        ctx = jnp.concatenate([tt_ref[...], ct_ref[...], et_ref[...],
                               pht_ref[...], ckt_ref[...], mt_ref[...]],
                              axis=0)
        ctx_scr[...] = ln(ctx)

    for s in range(_SQS):
        o = s * _STRIDE
        out_ref[o:o + 448] = ln(fused_scr[...] + sq_ref[s])

        m = mb_ref[s]
        mat = lax.dot_general(m, mw_ref[...], (((0,), (0,)), ((), ())),
                              preferred_element_type=jnp.float32)
        out_ref[o + _MAT_OFF:o + _CTX_OFF] = ln(mat + mvb_ref[...])

        out_ref[o + _CTX_OFF:o + _CTX_OFF + 132] = ctx_scr[...]
        out_ref[o + 644:o + _STRIDE] = jnp.zeros((4, _D), jnp.float32)


def _gather_body(table_hbm, idx_hbm, out_hbm, idx_v, bufs, gs0, gs1, ss0, ss1):
    wid = lax.axis_index("s") * 2 + lax.axis_index("c")
    base = wid * _BPW
    pltpu.sync_copy(idx_hbm.at[pl.ds(base * _PTOK, _BPW * _PTOK)], idx_v)
    gsem = (gs0, gs1)
    ssem = (ss0, ss1)

    def issue_gathers(g, p):
        for b in range(_NB):
            j = g * _NB + b
            pltpu.async_copy(
                table_hbm.at[idx_v.at[pl.ds(j * _PTOK, _NTOK)]],
                bufs.at[p, b], gsem[p])

    def drain_gathers(p):
        for b in range(_NB):
            pltpu.make_async_copy(out_hbm.at[base], bufs.at[p, b],
                                  gsem[p]).wait()

    def issue_store(g, p):
        pltpu.async_copy(bufs.at[p], out_hbm.at[pl.ds(base + g * _NB, _NB)],
                         ssem[p])

    def drain_store(p):
        pltpu.make_async_copy(bufs.at[p], out_hbm.at[pl.ds(base, _NB)],
                              ssem[p]).wait()

    issue_gathers(0, 0)

    def super_body(si, carry):
        g0 = si * 2

        @pl.when(si > 0)
        def _():
            drain_store(1)
        issue_gathers(g0 + 1, 1)
        drain_gathers(0)
        issue_store(g0, 0)
        drain_gathers(1)
        issue_store(g0 + 1, 1)

        @pl.when(si < _NSUP - 1)
        def _():
            drain_store(0)
            issue_gathers(g0 + 2, 0)
        return carry

    lax.fori_loop(0, _NSUP, super_body, 0)
    drain_store(0)
    drain_store(1)


def _sc_gather(table, idx_all):
    f = functools.partial(
        pl.kernel,
        mesh=plsc.VectorSubcoreMesh(core_axis_name="c", subcore_axis_name="s"),
        out_type=jax.ShapeDtypeStruct((_B, _NTOK, _D), jnp.float32),
        scratch_types=[
            pltpu.VMEM((_BPW * _PTOK,), jnp.int32),
            pltpu.VMEM((2, _NB, _NTOK, _D), jnp.float32),
            pltpu.SemaphoreType.DMA,
            pltpu.SemaphoreType.DMA,
            pltpu.SemaphoreType.DMA,
            pltpu.SemaphoreType.DMA,
        ],
    )(_gather_body)
    return f(table, idx_all)


@jax.jit
def kernel(piece_ids, color_ids, white_attacks, black_attacks, is_passed,
           is_isolated, is_doubled, material_balance, game_phase, is_check,
           mobility, turn, castling, ep, piece_table, color_W, color_b,
           square_table, turn_table, castling_table, ep_table, attack_w_table,
           attack_b_table, passed_table, isolated_table, doubled_table,
           material_W, material_b, phase_table, check_table, mobility_table,
           ln_gamma, ln_beta):
    full = lambda a: pl.BlockSpec(a.shape, lambda i: (0,) * a.ndim)

    sq_r = square_table.reshape(_NSQ, 1, _D)
    mb_r = material_balance.reshape(_NSQ, 1, 64)
    mw_r = material_W.reshape(1, _D)
    mvb_r = material_b.reshape(1, _D)
    g_r = ln_gamma.reshape(1, _D)
    b_r = ln_beta.reshape(1, _D)

    table = pl.pallas_call(
        _table_body,
        grid=(_TSTEPS,),
        in_specs=[
            full(piece_table), full(color_W), full(color_b),
            pl.BlockSpec((_SQS, 1, _D), lambda i: (i, 0, 0)),
            full(attack_w_table), full(attack_b_table), full(passed_table),
            full(isolated_table), full(doubled_table),
            full(turn_table), full(castling_table), full(ep_table),
            full(phase_table), full(check_table), full(mobility_table),
            pl.BlockSpec((_SQS, 1, 64), lambda i: (i, 0, 0)),
            full(mw_r), full(mvb_r), full(g_r), full(b_r),
        ],
        out_specs=pl.BlockSpec((_SQS * _STRIDE, _D), lambda i: (i, 0)),
        out_shape=jax.ShapeDtypeStruct((_NTOT, _D), jnp.float32),
        scratch_shapes=[
            pltpu.VMEM((448, _D), jnp.float32),
            pltpu.VMEM((132, _D), jnp.float32),
        ],
    )(piece_table, color_W, color_b, sq_r, attack_w_table, attack_b_table,
      passed_table, isolated_table, doubled_table, turn_table, castling_table,
      ep_table, phase_table, check_table, mobility_table, mb_r, mw_r, mvb_r,
      g_r, b_r)

    cidx = (piece_ids * 64 + color_ids * 32 + white_attacks * 16
            + black_attacks * 8 + is_passed * 4 + is_isolated * 2 + is_doubled)
    t_iota = jnp.arange(_NSQ, dtype=jnp.int32)[None, :]
    idx_sq = t_iota * _STRIDE + cidx
    b_iota = jnp.arange(_B, dtype=jnp.int32)
    blk_base = (b_iota // 64) * _STRIDE
    cbase = blk_base + _CTX_OFF
    idx_ctx = jnp.stack([
        cbase + turn,
        cbase + 2 + castling,
        cbase + 18 + ep,
        blk_base + _MAT_OFF + (b_iota % 64),
        cbase + 27 + game_phase,
        cbase + 30 + is_check,
        cbase + 32 + jnp.clip(mobility, 0, 99),
    ], axis=1)
    pad = jnp.zeros((_B, _PTOK - _NTOK), jnp.int32)
    idx_all = jnp.concatenate([idx_sq, idx_ctx, pad], axis=1).reshape(-1)
    idx_all = idx_all.astype(jnp.int32)

    return _sc_gather(table, idx_all)
